# baseline (speedup 1.0000x reference)
import functools

import jax
import jax.numpy as jnp
from jax import lax
from jax.experimental import pallas as pl
from jax.experimental.pallas import tpu as pltpu
from jax.experimental.pallas import tpu_sc as plsc

V = 10242
F3 = 3 * 20480
N_UNIQUE = 30720
B = 8
NCH = 64
K = B * NCH
OUTW = V + N_UNIQUE

NC = 2
NS = 16
L = 16
NW = NC * NS
JPT = N_UNIQUE // NW
CHUNK = 64
NCHUNK = JPT // CHUNK


def _tbl_body(x_ref, tbl_ref):
    xb = x_ref[...]
    t = jnp.concatenate([
        (xb[0, 0:64] + xb[0, 64:128]),
        (xb[1, 0:64] + xb[1, 64:128]),
    ], axis=0) * 0.25
    tbl_ref[...] = t.T


def _tc_tables(x):
    return pl.pallas_call(
        _tbl_body,
        grid=(B // 2,),
        in_specs=[pl.BlockSpec((2, 128, V), lambda b: (b, 1, 0))],
        out_specs=pl.BlockSpec((V, 128), lambda b: (0, b)),
        out_shape=jax.ShapeDtypeStruct((V, K), jnp.float32),
    )(x)


def _sc_gather(tbl, sep_flat, unique):
    mesh = plsc.VectorSubcoreMesh(core_axis_name="c", subcore_axis_name="s")

    @functools.partial(
        pl.kernel,
        out_type=jax.ShapeDtypeStruct((N_UNIQUE, K), jnp.float32),
        mesh=mesh,
        scratch_types=[
            pltpu.VMEM((JPT,), jnp.int32),
            pltpu.VMEM((JPT,), jnp.int32),
            pltpu.VMEM((CHUNK,), jnp.int32),
            pltpu.VMEM((CHUNK,), jnp.int32),
            pltpu.VMEM((CHUNK,), jnp.int32),
            pltpu.VMEM((CHUNK, K), jnp.float32),
            pltpu.VMEM((CHUNK, K), jnp.float32),
            pltpu.SemaphoreType.DMA,
            pltpu.SemaphoreType.DMA,
        ],
    )
    def k(tbl_hbm, sep_hbm, uq_hbm, out_hbm,
          ia_all, ib_all, uq_v, u2a, u2b, rows_a, rows_b, sem1, sem2):
        wid = lax.axis_index("s") * NC + lax.axis_index("c")
        jbase = wid * JPT

        for t in range(NCHUNK):
            pltpu.sync_copy(uq_hbm.at[pl.ds(jbase + t * CHUNK, CHUNK)], uq_v)
            for j in range(CHUNK // L):
                u2 = uq_v[pl.ds(j * L, L)] * 2
                u2a[pl.ds(j * L, L)] = u2
                u2b[pl.ds(j * L, L)] = u2 + 1
            ga = pltpu.async_copy(sep_hbm.at[u2a],
                                  ia_all.at[pl.ds(t * CHUNK, CHUNK)], sem1)
            gb = pltpu.async_copy(sep_hbm.at[u2b],
                                  ib_all.at[pl.ds(t * CHUNK, CHUNK)], sem2)
            ga.wait()
            gb.wait()

        for t in range(NCHUNK):
            j0 = jbase + t * CHUNK
            ia = ia_all.at[pl.ds(t * CHUNK, CHUNK)]
            ib = ib_all.at[pl.ds(t * CHUNK, CHUNK)]
            ca = pltpu.async_copy(tbl_hbm.at[ia], rows_a, sem1)
            cb = pltpu.async_copy(tbl_hbm.at[ib], rows_b, sem2)
            ca.wait()
            cb.wait()

            def add_row(r, carry):
                for u in range(K // L):
                    plsc.addupdate(rows_a.at[r, pl.ds(u * L, L)],
                                   rows_b[r, pl.ds(u * L, L)])
                return carry

            lax.fori_loop(0, CHUNK, add_row, 0, unroll=2)
            pltpu.sync_copy(rows_a, out_hbm.at[pl.ds(j0, CHUNK), :])

    return k(tbl, sep_flat, unique)


def _asm_body(x_ref, face_ref, out_ref):
    xb = x_ref[0]
    out_ref[0, :, 0:V] = (xb[0:64] + xb[64:128]) * 0.5
    out_ref[0, :, V:OUTW] = face_ref[...]


def _tc_assemble(x, face_t):
    return pl.pallas_call(
        _asm_body,
        grid=(B,),
        in_specs=[
            pl.BlockSpec((1, 128, V), lambda b: (b, 0, 0)),
            pl.BlockSpec((NCH, N_UNIQUE), lambda b: (b, 0)),
        ],
        out_specs=pl.BlockSpec((1, NCH, OUTW), lambda b: (b, 0, 0)),
        out_shape=jax.ShapeDtypeStruct((B, NCH, OUTW), jnp.float32),
    )(x, face_t)


def kernel(x, separated_src_idx, unique):
    tbl = _tc_tables(x)
    sep_flat = separated_src_idx.reshape(F3 * 2)
    outg = _sc_gather(tbl, sep_flat, unique)
    face_t = outg.T
    return _tc_assemble(x, face_t)


# speedup vs baseline: 1.0591x; 1.0591x over previous
import functools

import jax
import jax.numpy as jnp
from jax import lax
from jax.experimental import pallas as pl
from jax.experimental.pallas import tpu as pltpu
from jax.experimental.pallas import tpu_sc as plsc

V = 10242
F3 = 3 * 20480
N_UNIQUE = 30720
B = 8
NCH = 64
K = B * NCH
OUTW = V + N_UNIQUE

NC = 2
NS = 16
L = 16
NW = NC * NS
JPT = N_UNIQUE // NW
CHUNK = 64
NCHUNK = JPT // CHUNK


def _p1_body(x_ref, p1_ref):
    xb = x_ref[0]
    p1_ref[0] = (xb[0:64] + xb[64:128]) * 0.5


def _tc_part1(x):
    return pl.pallas_call(
        _p1_body,
        grid=(B,),
        in_specs=[pl.BlockSpec((1, 128, V), lambda b: (b, 0, 0))],
        out_specs=pl.BlockSpec((1, NCH, V), lambda b: (b, 0, 0)),
        out_shape=jax.ShapeDtypeStruct((B, NCH, V), jnp.float32),
    )(x)


def _tbl_body(x_ref, tbl_ref):
    xb = x_ref[...]
    t = jnp.concatenate([
        (xb[0, 0:64] + xb[0, 64:128]),
        (xb[1, 0:64] + xb[1, 64:128]),
    ], axis=0) * 0.25
    tbl_ref[...] = t.T


def _tc_tables(x):
    return pl.pallas_call(
        _tbl_body,
        grid=(B // 2,),
        in_specs=[pl.BlockSpec((2, 128, V), lambda b: (b, 1, 0))],
        out_specs=pl.BlockSpec((V, 128), lambda b: (0, b)),
        out_shape=jax.ShapeDtypeStruct((V, K), jnp.float32),
    )(x)


def _sc_gather(tbl, sep2d, unique):
    mesh = plsc.VectorSubcoreMesh(core_axis_name="c", subcore_axis_name="s")

    @functools.partial(
        pl.kernel,
        out_type=jax.ShapeDtypeStruct((N_UNIQUE, K), jnp.float32),
        mesh=mesh,
        scratch_types=[
            pltpu.VMEM((JPT,), jnp.int32),
            pltpu.VMEM((JPT,), jnp.int32),
            pltpu.VMEM((CHUNK,), jnp.int32),
            pltpu.VMEM((CHUNK,), jnp.int32),
            pltpu.VMEM((CHUNK,), jnp.int32),
            pltpu.VMEM((CHUNK, K), jnp.float32),
            pltpu.VMEM((CHUNK, K), jnp.float32),
            pltpu.SemaphoreType.DMA,
            pltpu.SemaphoreType.DMA,
        ],
    )
    def k(tbl_hbm, sep_hbm, uq_hbm, out_hbm,
          ia_all, ib_all, uq_v, u2a, u2b, rows_a, rows_b,
          sem1, sem2):
        wid = lax.axis_index("s") * NC + lax.axis_index("c")
        jbase = wid * JPT

        for t in range(NCHUNK):
            pltpu.sync_copy(uq_hbm.at[pl.ds(jbase + t * CHUNK, CHUNK)], uq_v)
            for j in range(CHUNK // L):
                u2 = uq_v[pl.ds(j * L, L)] * 2
                u2a[pl.ds(j * L, L)] = u2
                u2b[pl.ds(j * L, L)] = u2 + 1
            ga = pltpu.async_copy(sep_hbm.at[u2a],
                                  ia_all.at[pl.ds(t * CHUNK, CHUNK)], sem1)
            gb = pltpu.async_copy(sep_hbm.at[u2b],
                                  ib_all.at[pl.ds(t * CHUNK, CHUNK)], sem2)
            ga.wait()
            gb.wait()

        for t in range(NCHUNK):
            j0 = jbase + t * CHUNK
            ia = ia_all.at[pl.ds(t * CHUNK, CHUNK)]
            ib = ib_all.at[pl.ds(t * CHUNK, CHUNK)]
            ca = pltpu.async_copy(tbl_hbm.at[ia], rows_a, sem1)
            cb = pltpu.async_copy(tbl_hbm.at[ib], rows_b, sem2)
            ca.wait()
            cb.wait()

            def add_row(r, carry):
                for u in range(K // L):
                    plsc.addupdate(rows_a.at[r, pl.ds(u * L, L)],
                                   rows_b[r, pl.ds(u * L, L)])
                return carry

            lax.fori_loop(0, CHUNK, add_row, 0, unroll=2)
            pltpu.sync_copy(rows_a, out_hbm.at[pl.ds(j0, CHUNK), :])

    return k(tbl, sep2d, unique)


JH = N_UNIQUE // 2


def _asm_body(p1_ref, outg_ref, out_ref):
    h = pl.program_id(1)
    parity = pl.program_id(0) % 2
    sub = JH // 4

    @pl.when(h == 0)
    def _():
        out_ref[0, :, 0:V] = p1_ref[0]

    for par in range(2):
        r0 = par * 64

        @pl.when((h == 0) & (parity == par))
        def _():
            for c in range(4):
                tt = outg_ref[c * sub:(c + 1) * sub, :].T
                out_ref[0, :, V + c * sub:V + (c + 1) * sub] = (
                    tt[r0:r0 + 64, :])

        @pl.when((h == 1) & (parity == par))
        def _():
            for c in range(4):
                tt = outg_ref[c * sub:(c + 1) * sub, :].T
                out_ref[0, :, V + JH + c * sub:V + JH + (c + 1) * sub] = (
                    tt[r0:r0 + 64, :])


def _tc_assemble(p1, outg):
    return pl.pallas_call(
        _asm_body,
        grid=(B, 2),
        in_specs=[
            pl.BlockSpec((1, NCH, V), lambda b, h: (b, 0, 0)),
            pl.BlockSpec((JH, 128), lambda b, h: (h, b // 2)),
        ],
        out_specs=pl.BlockSpec((1, NCH, OUTW), lambda b, h: (b, 0, 0)),
        out_shape=jax.ShapeDtypeStruct((B, NCH, OUTW), jnp.float32),
    )(p1, outg)


def kernel(x, separated_src_idx, unique):
    p1 = _tc_part1(x)
    tbl = _tc_tables(x)
    sep_flat = separated_src_idx.reshape(F3 * 2)
    outg = _sc_gather(tbl, sep_flat, unique)
    return _tc_assemble(p1, outg)


# speedup vs baseline: 1.1850x; 1.1190x over previous
import functools

import jax
import jax.numpy as jnp
from jax import lax
from jax.experimental import pallas as pl
from jax.experimental.pallas import tpu as pltpu
from jax.experimental.pallas import tpu_sc as plsc

V = 10242
F3 = 3 * 20480
N_UNIQUE = 30720
B = 8
NCH = 64
K = B * NCH
OUTW = V + N_UNIQUE

NC = 2
NS = 16
L = 16
NW = NC * NS
JPT = N_UNIQUE // NW
CHUNK = 48
NCHUNK = JPT // CHUNK


def _p1_body(x_ref, p1_ref):
    xb = x_ref[0]
    p1_ref[0] = (xb[0:64] + xb[64:128]) * 0.5


def _tc_part1(x):
    return pl.pallas_call(
        _p1_body,
        grid=(B,),
        in_specs=[pl.BlockSpec((1, 128, V), lambda b: (b, 0, 0))],
        out_specs=pl.BlockSpec((1, NCH, V), lambda b: (b, 0, 0)),
        out_shape=jax.ShapeDtypeStruct((B, NCH, V), jnp.float32),
    )(x)


def _tbl_body(x_ref, tbl_ref):
    xb = x_ref[...]
    t = jnp.concatenate([
        (xb[0, 0:64] + xb[0, 64:128]),
        (xb[1, 0:64] + xb[1, 64:128]),
    ], axis=0) * 0.25
    tbl_ref[...] = t.T


def _tc_tables(x):
    return pl.pallas_call(
        _tbl_body,
        grid=(B // 2,),
        in_specs=[pl.BlockSpec((2, 128, V), lambda b: (b, 1, 0))],
        out_specs=pl.BlockSpec((V, 128), lambda b: (0, b)),
        out_shape=jax.ShapeDtypeStruct((V, K), jnp.float32),
    )(x)


def _sc_gather(tbl, sep2d, unique):
    mesh = plsc.VectorSubcoreMesh(core_axis_name="c", subcore_axis_name="s")

    @functools.partial(
        pl.kernel,
        out_type=jax.ShapeDtypeStruct((N_UNIQUE, K), jnp.float32),
        mesh=mesh,
        scratch_types=[
            pltpu.VMEM((JPT,), jnp.int32),
            pltpu.VMEM((JPT,), jnp.int32),
            pltpu.VMEM((CHUNK,), jnp.int32),
            pltpu.VMEM((CHUNK,), jnp.int32),
            pltpu.VMEM((CHUNK,), jnp.int32),
            pltpu.VMEM((CHUNK, K), jnp.float32),
            pltpu.VMEM((CHUNK, K), jnp.float32),
            pltpu.VMEM((CHUNK, K), jnp.float32),
            pltpu.VMEM((CHUNK, K), jnp.float32),
            pltpu.SemaphoreType.DMA,
            pltpu.SemaphoreType.DMA,
            pltpu.SemaphoreType.DMA,
            pltpu.SemaphoreType.DMA,
            pltpu.SemaphoreType.DMA,
            pltpu.SemaphoreType.DMA,
        ],
    )
    def k(tbl_hbm, sep_hbm, uq_hbm, out_hbm,
          ia_all, ib_all, uq_v, u2a, u2b, ra0, rb0, ra1, rb1,
          sga0, sgb0, sga1, sgb1, sw0, sw1):
        sem1, sem2 = sga0, sgb0
        wid = lax.axis_index("s") * NC + lax.axis_index("c")
        jbase = wid * JPT

        for t in range(NCHUNK):
            pltpu.sync_copy(uq_hbm.at[pl.ds(jbase + t * CHUNK, CHUNK)], uq_v)
            for j in range(CHUNK // L):
                u2 = uq_v[pl.ds(j * L, L)] * 2
                u2a[pl.ds(j * L, L)] = u2
                u2b[pl.ds(j * L, L)] = u2 + 1
            ga = pltpu.async_copy(sep_hbm.at[u2a],
                                  ia_all.at[pl.ds(t * CHUNK, CHUNK)], sem1)
            gb = pltpu.async_copy(sep_hbm.at[u2b],
                                  ib_all.at[pl.ds(t * CHUNK, CHUNK)], sem2)
            ga.wait()
            gb.wait()

        ra = (ra0, ra1)
        rb = (rb0, rb1)
        sg = ((sga0, sgb0), (sga1, sgb1))
        sw = (sw0, sw1)
        gd = {}
        wd = {}

        def fire_gather(t):
            s = t % 2
            ia = ia_all.at[pl.ds(t * CHUNK, CHUNK)]
            ib = ib_all.at[pl.ds(t * CHUNK, CHUNK)]
            gd[t] = (pltpu.async_copy(tbl_hbm.at[ia], ra[s], sg[s][0]),
                     pltpu.async_copy(tbl_hbm.at[ib], rb[s], sg[s][1]))

        fire_gather(0)
        for t in range(NCHUNK):
            s = t % 2
            if t + 1 < NCHUNK:
                if t - 1 >= 0:
                    wd[t - 1].wait()
                fire_gather(t + 1)
            ca, cb = gd[t]
            ca.wait()
            cb.wait()
            rows_a, rows_b = ra[s], rb[s]

            def add_row(r, carry):
                for u in range(K // L):
                    plsc.addupdate(rows_a.at[r, pl.ds(u * L, L)],
                                   rows_b[r, pl.ds(u * L, L)])
                return carry

            lax.fori_loop(0, CHUNK, add_row, 0, unroll=2)
            wd[t] = pltpu.async_copy(
                rows_a, out_hbm.at[pl.ds(jbase + t * CHUNK, CHUNK), :], sw[s])
        wd[NCHUNK - 2].wait()
        wd[NCHUNK - 1].wait()

    return k(tbl, sep2d, unique)


JH = N_UNIQUE // 2


def _asm_body(p1_ref, outg_ref, out_ref):
    h = pl.program_id(1)
    parity = pl.program_id(0) % 2
    sub = JH // 4

    @pl.when(h == 0)
    def _():
        out_ref[0, :, 0:V] = p1_ref[0]

    for par in range(2):
        r0 = par * 64

        @pl.when((h == 0) & (parity == par))
        def _():
            for c in range(4):
                tt = outg_ref[c * sub:(c + 1) * sub, :].T
                out_ref[0, :, V + c * sub:V + (c + 1) * sub] = (
                    tt[r0:r0 + 64, :])

        @pl.when((h == 1) & (parity == par))
        def _():
            for c in range(4):
                tt = outg_ref[c * sub:(c + 1) * sub, :].T
                out_ref[0, :, V + JH + c * sub:V + JH + (c + 1) * sub] = (
                    tt[r0:r0 + 64, :])


def _tc_assemble(p1, outg):
    return pl.pallas_call(
        _asm_body,
        grid=(B, 2),
        in_specs=[
            pl.BlockSpec((1, NCH, V), lambda b, h: (b, 0, 0)),
            pl.BlockSpec((JH, 128), lambda b, h: (h, b // 2)),
        ],
        out_specs=pl.BlockSpec((1, NCH, OUTW), lambda b, h: (b, 0, 0)),
        out_shape=jax.ShapeDtypeStruct((B, NCH, OUTW), jnp.float32),
    )(p1, outg)


def kernel(x, separated_src_idx, unique):
    p1 = _tc_part1(x)
    tbl = _tc_tables(x)
    sep_flat = separated_src_idx.reshape(F3 * 2)
    outg = _sc_gather(tbl, sep_flat, unique)
    return _tc_assemble(p1, outg)


# speedup vs baseline: 1.1864x; 1.0011x over previous
import functools

import jax
import jax.numpy as jnp
from jax import lax
from jax.experimental import pallas as pl
from jax.experimental.pallas import tpu as pltpu
from jax.experimental.pallas import tpu_sc as plsc

V = 10242
F3 = 3 * 20480
N_UNIQUE = 30720
B = 8
NCH = 64
K = B * NCH
OUTW = V + N_UNIQUE

NC = 2
NS = 16
L = 16
NW = NC * NS
JPT = N_UNIQUE // NW
CHUNK = 48
NCHUNK = JPT // CHUNK


def _p1_body(x_ref, p1_ref):
    xb = x_ref[0]
    p1_ref[0] = (xb[0:64] + xb[64:128]) * 0.5


def _tc_part1(x):
    return pl.pallas_call(
        _p1_body,
        grid=(B,),
        in_specs=[pl.BlockSpec((1, 128, V), lambda b: (b, 0, 0))],
        out_specs=pl.BlockSpec((1, NCH, V), lambda b: (b, 0, 0)),
        out_shape=jax.ShapeDtypeStruct((B, NCH, V), jnp.float32),
    )(x)


def _tbl_body(x_ref, tbl_ref):
    xb = x_ref[...]
    t = jnp.concatenate([
        (xb[0, 0:64] + xb[0, 64:128]),
        (xb[1, 0:64] + xb[1, 64:128]),
    ], axis=0) * 0.25
    tbl_ref[...] = t.T


def _tc_tables(x):
    return pl.pallas_call(
        _tbl_body,
        grid=(B // 2,),
        in_specs=[pl.BlockSpec((2, 128, V), lambda b: (b, 1, 0))],
        out_specs=pl.BlockSpec((V, 128), lambda b: (0, b)),
        out_shape=jax.ShapeDtypeStruct((V, K), jnp.float32),
    )(x)


def _sc_gather(tbl, sep_flat, unique):
    mesh = plsc.VectorSubcoreMesh(core_axis_name="c", subcore_axis_name="s")

    @functools.partial(
        pl.kernel,
        out_type=jax.ShapeDtypeStruct((N_UNIQUE, K), jnp.float32),
        mesh=mesh,
        scratch_types=[
            pltpu.VMEM((JPT,), jnp.int32),
            pltpu.VMEM((JPT,), jnp.int32),
            pltpu.VMEM((CHUNK,), jnp.int32),
            pltpu.VMEM((CHUNK,), jnp.int32),
            pltpu.VMEM((CHUNK,), jnp.int32),
            pltpu.VMEM((CHUNK, K), jnp.float32),
            pltpu.VMEM((CHUNK, K), jnp.float32),
            pltpu.VMEM((CHUNK, K), jnp.float32),
            pltpu.VMEM((CHUNK, K), jnp.float32),
            pltpu.SemaphoreType.DMA,
            pltpu.SemaphoreType.DMA,
            pltpu.SemaphoreType.DMA,
            pltpu.SemaphoreType.DMA,
            pltpu.SemaphoreType.DMA,
            pltpu.SemaphoreType.DMA,
        ],
    )
    def k(tbl_hbm, sep_hbm, uq_hbm, out_hbm,
          ia_all, ib_all, uq_v, u2a, u2b, ra0, rb0, ra1, rb1,
          sga0, sgb0, sga1, sgb1, sw0, sw1):
        sem1, sem2 = sga0, sgb0
        wid = lax.axis_index("s") * NC + lax.axis_index("c")
        jbase = wid * JPT

        for t in range(NCHUNK):
            pltpu.sync_copy(uq_hbm.at[pl.ds(jbase + t * CHUNK, CHUNK)], uq_v)
            for j in range(CHUNK // L):
                u2 = uq_v[pl.ds(j * L, L)] * 2
                u2a[pl.ds(j * L, L)] = u2
                u2b[pl.ds(j * L, L)] = u2 + 1
            ga = pltpu.async_copy(sep_hbm.at[u2a],
                                  ia_all.at[pl.ds(t * CHUNK, CHUNK)], sem1)
            gb = pltpu.async_copy(sep_hbm.at[u2b],
                                  ib_all.at[pl.ds(t * CHUNK, CHUNK)], sem2)
            ga.wait()
            gb.wait()

        ra = (ra0, ra1)
        rb = (rb0, rb1)
        sg = ((sga0, sgb0), (sga1, sgb1))
        sw = (sw0, sw1)
        gd = {}
        wd = {}

        def fire_gather(t):
            s = t % 2
            ia = ia_all.at[pl.ds(t * CHUNK, CHUNK)]
            ib = ib_all.at[pl.ds(t * CHUNK, CHUNK)]
            gd[t] = (pltpu.async_copy(tbl_hbm.at[ia], ra[s], sg[s][0]),
                     pltpu.async_copy(tbl_hbm.at[ib], rb[s], sg[s][1]))

        fire_gather(0)
        for t in range(NCHUNK):
            s = t % 2
            if t + 1 < NCHUNK:
                if t - 1 >= 0:
                    wd[t - 1].wait()
                fire_gather(t + 1)
            ca, cb = gd[t]
            ca.wait()
            cb.wait()
            rows_a, rows_b = ra[s], rb[s]

            def add_row(r, carry):
                for u in range(K // L):
                    plsc.addupdate(rows_a.at[r, pl.ds(u * L, L)],
                                   rows_b[r, pl.ds(u * L, L)])
                return carry

            lax.fori_loop(0, CHUNK, add_row, 0, unroll=2)
            wd[t] = pltpu.async_copy(
                rows_a, out_hbm.at[pl.ds(jbase + t * CHUNK, CHUNK), :], sw[s])
        wd[NCHUNK - 2].wait()
        wd[NCHUNK - 1].wait()

    return k(tbl, sep_flat, unique)


JH = N_UNIQUE // 2


def _asm_body(p1_ref, outg_ref, out_ref):
    h = pl.program_id(1)
    parity = pl.program_id(0) % 2
    sub = JH // 4

    @pl.when(h == 0)
    def _():
        out_ref[0, :, 0:V] = p1_ref[0]

    for par in range(2):
        r0 = par * 64

        @pl.when((h == 0) & (parity == par))
        def _():
            for c in range(4):
                tt = outg_ref[c * sub:(c + 1) * sub, :].T
                out_ref[0, :, V + c * sub:V + (c + 1) * sub] = (
                    tt[r0:r0 + 64, :])

        @pl.when((h == 1) & (parity == par))
        def _():
            for c in range(4):
                tt = outg_ref[c * sub:(c + 1) * sub, :].T
                out_ref[0, :, V + JH + c * sub:V + JH + (c + 1) * sub] = (
                    tt[r0:r0 + 64, :])


def _tc_assemble(p1, outg):
    return pl.pallas_call(
        _asm_body,
        grid=(B, 2),
        in_specs=[
            pl.BlockSpec((1, NCH, V), lambda b, h: (b, 0, 0)),
            pl.BlockSpec((JH, 128), lambda b, h: (h, b // 2)),
        ],
        out_specs=pl.BlockSpec((1, NCH, OUTW), lambda b, h: (b, 0, 0)),
        out_shape=jax.ShapeDtypeStruct((B, NCH, OUTW), jnp.float32),
    )(p1, outg)


def kernel(x, separated_src_idx, unique):
    p1 = _tc_part1(x)
    tbl = _tc_tables(x)
    sep_flat = separated_src_idx.reshape(F3 * 2)
    outg = _sc_gather(tbl, sep_flat, unique)
    return _tc_assemble(p1, outg)
